# Initial kernel scaffold; baseline (speedup 1.0000x reference)
import jax
import jax.numpy as jnp
from jax.experimental import pallas as pl


def kernel(input, projector, codebook):
    raise NotImplementedError("write your pallas kernel here")


# speedup vs baseline: 1.4297x; 1.4297x over previous
import jax
import jax.numpy as jnp
from jax.experimental import pallas as pl

_INPUT_DIM = 256
_EMBED_DIM = 64
_NUM_EMBED = 1024
_CODEBOOK_NUM = 4


def _vq_kernel(inp_ref, proj_ref, cb_ref, out_ref):
    xb = inp_ref[...].astype(jnp.bfloat16)
    pb = proj_ref[...].astype(jnp.bfloat16)
    x = jnp.dot(xb, pb, preferred_element_type=jnp.float32)
    xn = x / jnp.sqrt(jnp.sum(x * x, axis=1, keepdims=True))
    xnb = xn.astype(jnp.bfloat16)
    cb = cb_ref[...]
    cbn = cb / jnp.sqrt(jnp.sum(cb * cb, axis=1, keepdims=True))
    cbnb = cbn.astype(jnp.bfloat16)
    cols = []
    for c in range(_CODEBOOK_NUM):
        cbc = cbnb[c * _NUM_EMBED:(c + 1) * _NUM_EMBED, :]
        s = jax.lax.dot_general(
            xnb, cbc, (((1,), (1,)), ((), ())),
            preferred_element_type=jnp.float32)
        cols.append(jnp.argmax(s, axis=1).astype(jnp.int32)[:, None])
    out_ref[...] = jnp.concatenate(cols, axis=1)


def kernel(input, projector, codebook):
    b, n, t, d = input.shape
    tokens = b * n * t
    inp2 = input.reshape(tokens, d)
    cb2 = codebook.reshape(_CODEBOOK_NUM * _NUM_EMBED, _EMBED_DIM)
    tile = 1024
    out = pl.pallas_call(
        _vq_kernel,
        grid=(tokens // tile,),
        in_specs=[
            pl.BlockSpec((tile, d), lambda i: (i, 0)),
            pl.BlockSpec((d, _EMBED_DIM), lambda i: (0, 0)),
            pl.BlockSpec((_CODEBOOK_NUM * _NUM_EMBED, _EMBED_DIM), lambda i: (0, 0)),
        ],
        out_specs=pl.BlockSpec((tile, _CODEBOOK_NUM), lambda i: (i, 0)),
        out_shape=jax.ShapeDtypeStruct((tokens, _CODEBOOK_NUM), jnp.int32),
    )(inp2, projector, cb2)
    return out.reshape(b, n, t, _CODEBOOK_NUM)


# speedup vs baseline: 1.8889x; 1.3211x over previous
import jax
import jax.numpy as jnp
from jax.experimental import pallas as pl
from jax.experimental.pallas import tpu as pltpu

_INPUT_DIM = 256
_EMBED_DIM = 64
_NUM_EMBED = 1024
_CODEBOOK_NUM = 4


def _vq_kernel(inp_ref, proj_ref, cb_ref, out_ref, cbn_ref):
    @pl.when(pl.program_id(0) == 0)
    def _():
        cb = cb_ref[...]
        cbn = cb / jnp.sqrt(jnp.sum(cb * cb, axis=1, keepdims=True))
        cbn_ref[...] = cbn.astype(jnp.bfloat16)

    xb = inp_ref[...].astype(jnp.bfloat16)
    pb = proj_ref[...].astype(jnp.bfloat16)
    x = jnp.dot(xb, pb, preferred_element_type=jnp.float32)
    xn = x / jnp.sqrt(jnp.sum(x * x, axis=1, keepdims=True))
    xnb = xn.astype(jnp.bfloat16)
    cbnb = cbn_ref[...]
    cols = []
    for c in range(_CODEBOOK_NUM):
        cbc = cbnb[c * _NUM_EMBED:(c + 1) * _NUM_EMBED, :]
        s = jax.lax.dot_general(
            xnb, cbc, (((1,), (1,)), ((), ())),
            preferred_element_type=jnp.float32)
        cols.append(jnp.argmax(s, axis=1).astype(jnp.int32)[:, None])
    out_ref[...] = jnp.concatenate(cols, axis=1)


def kernel(input, projector, codebook):
    b, n, t, d = input.shape
    tokens = b * n * t
    inp2 = input.reshape(tokens, d)
    cb2 = codebook.reshape(_CODEBOOK_NUM * _NUM_EMBED, _EMBED_DIM)
    tile = 1024
    out = pl.pallas_call(
        _vq_kernel,
        grid=(tokens // tile,),
        in_specs=[
            pl.BlockSpec((tile, d), lambda i: (i, 0)),
            pl.BlockSpec((d, _EMBED_DIM), lambda i: (0, 0)),
            pl.BlockSpec((_CODEBOOK_NUM * _NUM_EMBED, _EMBED_DIM), lambda i: (0, 0)),
        ],
        out_specs=pl.BlockSpec((tile, _CODEBOOK_NUM), lambda i: (i, 0)),
        out_shape=jax.ShapeDtypeStruct((tokens, _CODEBOOK_NUM), jnp.int32),
        scratch_shapes=[pltpu.VMEM((_CODEBOOK_NUM * _NUM_EMBED, _EMBED_DIM),
                                   jnp.bfloat16)],
    )(inp2, projector, cb2)
    return out.reshape(b, n, t, _CODEBOOK_NUM)


# speedup vs baseline: 1.9618x; 1.0386x over previous
import jax
import jax.numpy as jnp
from jax.experimental import pallas as pl
from jax.experimental.pallas import tpu as pltpu

_INPUT_DIM = 256
_EMBED_DIM = 64
_NUM_EMBED = 1024
_CODEBOOK_NUM = 4


def _vq_kernel(inp_ref, proj_ref, cb_ref, out_ref, cbn_ref):
    @pl.when(pl.program_id(0) == 0)
    def _():
        cb = cb_ref[...]
        cbn = cb / jnp.sqrt(jnp.sum(cb * cb, axis=1, keepdims=True))
        cbn_ref[...] = cbn.astype(jnp.bfloat16)

    xb = inp_ref[...].astype(jnp.bfloat16)
    pb = proj_ref[...].astype(jnp.bfloat16)
    x = jnp.dot(xb, pb, preferred_element_type=jnp.float32)
    xn = x / jnp.sqrt(jnp.sum(x * x, axis=1, keepdims=True))
    xnb = xn.astype(jnp.bfloat16)
    cbnb = cbn_ref[...]
    cols = []
    for c in range(_CODEBOOK_NUM):
        cbc = cbnb[c * _NUM_EMBED:(c + 1) * _NUM_EMBED, :]
        s = jax.lax.dot_general(
            xnb, cbc, (((1,), (1,)), ((), ())),
            preferred_element_type=jnp.float32)
        cols.append(jnp.argmax(s, axis=1).astype(jnp.int32)[:, None])
    out_ref[...] = jnp.concatenate(cols, axis=1)


def kernel(input, projector, codebook):
    b, n, t, d = input.shape
    tokens = b * n * t
    inp2 = input.reshape(tokens, d)
    cb2 = codebook.reshape(_CODEBOOK_NUM * _NUM_EMBED, _EMBED_DIM)
    tile = 2048
    out = pl.pallas_call(
        _vq_kernel,
        grid=(tokens // tile,),
        in_specs=[
            pl.BlockSpec((tile, d), lambda i: (i, 0)),
            pl.BlockSpec((d, _EMBED_DIM), lambda i: (0, 0)),
            pl.BlockSpec((_CODEBOOK_NUM * _NUM_EMBED, _EMBED_DIM), lambda i: (0, 0)),
        ],
        out_specs=pl.BlockSpec((tile, _CODEBOOK_NUM), lambda i: (i, 0)),
        out_shape=jax.ShapeDtypeStruct((tokens, _CODEBOOK_NUM), jnp.int32),
        scratch_shapes=[pltpu.VMEM((_CODEBOOK_NUM * _NUM_EMBED, _EMBED_DIM),
                                   jnp.bfloat16)],
    )(inp2, projector, cb2)
    return out.reshape(b, n, t, _CODEBOOK_NUM)


# speedup vs baseline: 2.0402x; 1.0399x over previous
import jax
import jax.numpy as jnp
from jax.experimental import pallas as pl
from jax.experimental.pallas import tpu as pltpu

_INPUT_DIM = 256
_EMBED_DIM = 64
_NUM_EMBED = 1024
_CODEBOOK_NUM = 4


def _vq_kernel(inp_ref, proj_ref, cb_ref, out_ref, cbn_ref):
    @pl.when(pl.program_id(0) == 0)
    def _():
        cb = cb_ref[...]
        cbn = cb / jnp.sqrt(jnp.sum(cb * cb, axis=1, keepdims=True))
        cbn_ref[...] = cbn.astype(jnp.bfloat16)

    xb = inp_ref[...].astype(jnp.bfloat16)
    pb = proj_ref[...].astype(jnp.bfloat16)
    x = jnp.dot(xb, pb, preferred_element_type=jnp.float32)
    xn = x * jax.lax.rsqrt(jnp.sum(x * x, axis=1, keepdims=True))
    xnb = xn.astype(jnp.bfloat16)
    cbnb = cbn_ref[...]
    cols = []
    for c in range(_CODEBOOK_NUM):
        cbc = cbnb[c * _NUM_EMBED:(c + 1) * _NUM_EMBED, :]
        s = jax.lax.dot_general(
            xnb, cbc, (((1,), (1,)), ((), ())),
            preferred_element_type=jnp.float32)
        cols.append(jnp.argmax(s, axis=1).astype(jnp.int32)[:, None])
    out_ref[...] = jnp.concatenate(cols, axis=1)


def kernel(input, projector, codebook):
    b, n, t, d = input.shape
    tokens = b * n * t
    inp2 = input.reshape(tokens, d)
    cb2 = codebook.reshape(_CODEBOOK_NUM * _NUM_EMBED, _EMBED_DIM)
    tile = 2048
    out = pl.pallas_call(
        _vq_kernel,
        grid=(tokens // tile,),
        in_specs=[
            pl.BlockSpec((tile, d), lambda i: (i, 0)),
            pl.BlockSpec((d, _EMBED_DIM), lambda i: (0, 0)),
            pl.BlockSpec((_CODEBOOK_NUM * _NUM_EMBED, _EMBED_DIM), lambda i: (0, 0)),
        ],
        out_specs=pl.BlockSpec((tile, _CODEBOOK_NUM), lambda i: (i, 0)),
        out_shape=jax.ShapeDtypeStruct((tokens, _CODEBOOK_NUM), jnp.int32),
        scratch_shapes=[pltpu.VMEM((_CODEBOOK_NUM * _NUM_EMBED, _EMBED_DIM),
                                   jnp.bfloat16)],
    )(inp2, projector, cb2)
    return out.reshape(b, n, t, _CODEBOOK_NUM)


# speedup vs baseline: 2.0707x; 1.0150x over previous
import jax
import jax.numpy as jnp
from jax.experimental import pallas as pl
from jax.experimental.pallas import tpu as pltpu

_INPUT_DIM = 256
_EMBED_DIM = 64
_NUM_EMBED = 1024
_CODEBOOK_NUM = 4


def _vq_kernel(inp_ref, proj_ref, cb_ref, out_ref, cbn_ref):
    @pl.when(pl.program_id(0) == 0)
    def _():
        cb = cb_ref[...]
        cbn = cb / jnp.sqrt(jnp.sum(cb * cb, axis=1, keepdims=True))
        cbn_ref[...] = cbn.astype(jnp.bfloat16)

    xb = inp_ref[...].astype(jnp.bfloat16)
    pb = proj_ref[...].astype(jnp.bfloat16)
    x = jnp.dot(xb, pb, preferred_element_type=jnp.float32)
    xn = x * jax.lax.rsqrt(jnp.sum(x * x, axis=1, keepdims=True))
    xnb = xn.astype(jnp.bfloat16)
    cbnb = cbn_ref[...]
    cols = []
    for c in range(_CODEBOOK_NUM):
        cbc = cbnb[c * _NUM_EMBED:(c + 1) * _NUM_EMBED, :]
        s = jax.lax.dot_general(
            xnb, cbc, (((1,), (1,)), ((), ())),
            preferred_element_type=jnp.float32)
        cols.append(jnp.argmax(s, axis=1).astype(jnp.int32)[:, None])
    out_ref[...] = jnp.concatenate(cols, axis=1)


def kernel(input, projector, codebook):
    b, n, t, d = input.shape
    tokens = b * n * t
    inp2 = input.reshape(tokens, d)
    cb2 = codebook.reshape(_CODEBOOK_NUM * _NUM_EMBED, _EMBED_DIM)
    tile = 3584
    out = pl.pallas_call(
        _vq_kernel,
        grid=(tokens // tile,),
        in_specs=[
            pl.BlockSpec((tile, d), lambda i: (i, 0)),
            pl.BlockSpec((d, _EMBED_DIM), lambda i: (0, 0)),
            pl.BlockSpec((_CODEBOOK_NUM * _NUM_EMBED, _EMBED_DIM), lambda i: (0, 0)),
        ],
        out_specs=pl.BlockSpec((tile, _CODEBOOK_NUM), lambda i: (i, 0)),
        out_shape=jax.ShapeDtypeStruct((tokens, _CODEBOOK_NUM), jnp.int32),
        scratch_shapes=[pltpu.VMEM((_CODEBOOK_NUM * _NUM_EMBED, _EMBED_DIM),
                                   jnp.bfloat16)],
    )(inp2, projector, cb2)
    return out.reshape(b, n, t, _CODEBOOK_NUM)
